# Initial kernel scaffold; baseline (speedup 1.0000x reference)
import jax
import jax.numpy as jnp
from jax.experimental import pallas as pl


def kernel(wav):
    raise NotImplementedError("write your pallas kernel here")


# speedup vs baseline: 3.6792x; 3.6792x over previous
import functools

import jax
import jax.numpy as jnp
from jax import lax
from jax.experimental import pallas as pl
from jax.experimental.pallas import tpu as pltpu
from jax.experimental.pallas import tpu_sc as plsc

_SHIFT = 8192


def _build_shift_kernel(rows, time, length, num_workers, rows_per_worker):
    mesh = plsc.VectorSubcoreMesh(core_axis_name="c", subcore_axis_name="s")

    chunk = 30720
    assert length % chunk == 0
    nchunks = length // chunk

    @functools.partial(
        pl.kernel,
        mesh=mesh,
        out_type=jax.ShapeDtypeStruct((rows, length), jnp.float32),
        scratch_types=[
            pltpu.VMEM((rows + 16,), jnp.int32),
            pltpu.VMEM((chunk + 8,), jnp.float32),
            pltpu.VMEM((chunk,), jnp.float32),
        ],
        compiler_params=pltpu.CompilerParams(use_tc_tiling_on_sc=False),
    )
    def shift_kernel(wav_hbm, offs_hbm, out_hbm, offs_v, buf_in, buf_out):
        wid = lax.axis_index("s") * 2 + lax.axis_index("c")
        base_row = wid * rows_per_worker
        pltpu.sync_copy(offs_hbm, offs_v)
        offs_vec = offs_v[pl.ds(base_row, 16)]
        for r in range(rows_per_worker):
            row = base_row + r
            off = offs_vec[r]
            q8 = pl.multiple_of((off // 8) * 8, 8)
            rem = off - q8
            for c in range(nchunks):
                pltpu.sync_copy(
                    wav_hbm.at[row, pl.ds(q8 + c * chunk, chunk + 8)], buf_in
                )

                def shift_body(i, carry):
                    buf_out[pl.ds(i * 16, 16)] = buf_in[pl.ds(rem + i * 16, 16)]
                    return carry

                lax.fori_loop(0, chunk // 16, shift_body, 0)
                pltpu.sync_copy(buf_out, out_hbm.at[row, pl.ds(c * chunk, chunk)])

    return shift_kernel


def kernel(wav):
    batch, sources, channels, time = wav.shape
    length = time - _SHIFT
    rows = batch * sources * channels

    offs_key = jax.random.key(42)
    offsets = jax.random.randint(offs_key, (batch, sources, 1, 1), 0, _SHIFT)
    offs = jnp.broadcast_to(
        offsets.reshape(batch * sources, 1), (batch * sources, channels)
    ).reshape(rows).astype(jnp.int32)
    offs = jnp.pad(offs, (0, 16))

    num_workers = 32
    rows_per_worker = rows // num_workers
    wav2 = wav.reshape(rows, time)
    out2 = _build_shift_kernel(rows, time, length, num_workers, rows_per_worker)(
        wav2, offs
    )
    return out2.reshape(batch, sources, channels, length)


# speedup vs baseline: 6.3323x; 1.7211x over previous
import functools

import jax
import jax.numpy as jnp
from jax import lax
from jax.experimental import pallas as pl
from jax.experimental.pallas import tpu as pltpu
from jax.experimental.pallas import tpu_sc as plsc

_SHIFT = 8192
_NUM_WORKERS = 32
_CHUNK = 30720


def _build_shift_kernel(rows, time, length, rows_per_worker):
    mesh = plsc.VectorSubcoreMesh(core_axis_name="c", subcore_axis_name="s")
    chunk = _CHUNK
    assert length % chunk == 0
    nchunks = length // chunk
    nsteps = rows_per_worker * nchunks

    @functools.partial(
        pl.kernel,
        mesh=mesh,
        out_type=jax.ShapeDtypeStruct((rows, length), jnp.float32),
        scratch_types=[
            pltpu.VMEM((rows + 16,), jnp.int32),
            pltpu.VMEM((chunk + 8,), jnp.float32),
            pltpu.VMEM((chunk + 8,), jnp.float32),
            pltpu.VMEM((chunk,), jnp.float32),
            pltpu.VMEM((chunk,), jnp.float32),
            pltpu.SemaphoreType.DMA,
            pltpu.SemaphoreType.DMA,
            pltpu.SemaphoreType.DMA,
            pltpu.SemaphoreType.DMA,
        ],
        compiler_params=pltpu.CompilerParams(use_tc_tiling_on_sc=False),
    )
    def shift_kernel(
        wav_hbm, offs_hbm, out_hbm,
        offs_v, bin0, bin1, bout0, bout1, si0, si1, so0, so1,
    ):
        wid = lax.axis_index("s") * 2 + lax.axis_index("c")
        base_row = wid * rows_per_worker
        pltpu.sync_copy(offs_hbm, offs_v)
        offs_vec = offs_v[pl.ds(base_row, 16)]
        offs = [offs_vec[r] for r in range(rows_per_worker)]
        q8s = [pl.multiple_of((o // 8) * 8, 8) for o in offs]
        rems = [o - q for o, q in zip(offs, q8s)]

        bins = [bin0, bin1]
        bouts = [bout0, bout1]
        isems = [si0, si1]
        osems = [so0, so1]

        def in_copy(g):
            r, c = divmod(g, nchunks)
            return pltpu.async_copy(
                wav_hbm.at[base_row + r, pl.ds(q8s[r] + c * chunk, chunk + 8)],
                bins[g % 2],
                isems[g % 2],
            )

        def out_copy(g):
            r, c = divmod(g, nchunks)
            return pltpu.async_copy(
                bouts[g % 2],
                out_hbm.at[base_row + r, pl.ds(c * chunk, chunk)],
                osems[g % 2],
            )

        in_handles = {0: in_copy(0)}
        out_handles = {}
        for g in range(nsteps):
            if g + 1 < nsteps:
                in_handles[g + 1] = in_copy(g + 1)
            in_handles[g].wait()
            if g >= 2:
                out_handles[g - 2].wait()
            r, c = divmod(g, nchunks)
            rem = rems[r]
            bi = bins[g % 2]
            bo = bouts[g % 2]

            @plsc.parallel_loop(0, chunk // 16, unroll=8)
            def _shift(i):
                bo[pl.ds(i * 16, 16)] = bi[pl.ds(rem + i * 16, 16)]

            out_handles[g] = out_copy(g)
        out_handles[nsteps - 2].wait()
        out_handles[nsteps - 1].wait()

    return shift_kernel


def kernel(wav):
    batch, sources, channels, time = wav.shape
    length = time - _SHIFT
    rows = batch * sources * channels

    offs_key = jax.random.key(42)
    offsets = jax.random.randint(offs_key, (batch, sources, 1, 1), 0, _SHIFT)
    offs = jnp.broadcast_to(
        offsets.reshape(batch * sources, 1), (batch * sources, channels)
    ).reshape(rows).astype(jnp.int32)
    offs = jnp.pad(offs, (0, 16))

    rows_per_worker = rows // _NUM_WORKERS
    wav2 = wav.reshape(rows, time)
    out2 = _build_shift_kernel(rows, time, length, rows_per_worker)(wav2, offs)
    return out2.reshape(batch, sources, channels, length)


# speedup vs baseline: 17.0170x; 2.6873x over previous
import functools

import jax
import jax.numpy as jnp
from jax import lax
from jax.experimental import pallas as pl
from jax.experimental.pallas import tpu as pltpu
from jax.experimental.pallas import tpu_sc as plsc

_SHIFT = 8192
_NUM_WORKERS = 32
_SEGS = 120


def _build_shift_kernel(nblocks, in_tb, out_tb, channels, lanes, blocks_per_worker):
    mesh = plsc.VectorSubcoreMesh(core_axis_name="c", subcore_axis_name="s")
    segs = _SEGS
    assert out_tb % segs == 0
    nchunks = out_tb // segs
    nsteps = blocks_per_worker * nchunks

    @functools.partial(
        pl.kernel,
        mesh=mesh,
        out_type=jax.ShapeDtypeStruct((nblocks, out_tb, channels, lanes), jnp.float32),
        scratch_types=[
            pltpu.VMEM((nblocks + 16,), jnp.int32),
            pltpu.VMEM((segs + 1, channels, lanes), jnp.float32),
            pltpu.VMEM((segs + 1, channels, lanes), jnp.float32),
            pltpu.VMEM((segs, channels, lanes), jnp.float32),
            pltpu.VMEM((segs, channels, lanes), jnp.float32),
            pltpu.SemaphoreType.DMA,
            pltpu.SemaphoreType.DMA,
            pltpu.SemaphoreType.DMA,
            pltpu.SemaphoreType.DMA,
        ],
        compiler_params=pltpu.CompilerParams(
            use_tc_tiling_on_sc=False, needs_layout_passes=False
        ),
    )
    def shift_kernel(
        x_hbm, offs_hbm, out_hbm,
        offs_v, bin0, bin1, bout0, bout1, si0, si1, so0, so1,
    ):
        wid = lax.axis_index("s") * 2 + lax.axis_index("c")
        base_blk = wid * blocks_per_worker
        pltpu.sync_copy(offs_hbm, offs_v)
        offs_vec = offs_v[pl.ds(base_blk, 16)]
        lane = jax.lax.iota(jnp.int32, 16)
        qs, rems = [], []
        for b in range(blocks_per_worker):
            off = offs_vec[b]
            q = off // lanes
            qs.append(q)
            rems.append(off - q * lanes)
        bins = [bin0, bin1]
        bouts = [bout0, bout1]
        isems = [si0, si1]
        osems = [so0, so1]

        def in_copy(g):
            b, c = divmod(g, nchunks)
            return pltpu.async_copy(
                x_hbm.at[base_blk + b, pl.ds(qs[b] + c * segs, segs + 1), :, :],
                bins[g % 2],
                isems[g % 2],
            )

        def out_copy(g):
            b, c = divmod(g, nchunks)
            return pltpu.async_copy(
                bouts[g % 2],
                out_hbm.at[base_blk + b, pl.ds(c * segs, segs), :, :],
                osems[g % 2],
            )

        in_handles = {0: in_copy(0)}
        out_handles = {}
        for g in range(nsteps):
            if g + 1 < nsteps:
                in_handles[g + 1] = in_copy(g + 1)
            in_handles[g].wait()
            if g >= 2:
                out_handles[g - 2].wait()
            b, c = divmod(g, nchunks)
            rem = rems[b]
            bi = bins[g % 2]
            bo = bouts[g % 2]
            carries, wmods = [], []
            lane_bits = lanes.bit_length() - 1
            for jg in range(lanes // 16):
                w = rem + jg * 16 + lane
                carries.append(lax.shift_right_logical(w, lane_bits))
                wmods.append(jnp.bitwise_and(w, lanes - 1))
            cvecs = [jnp.full((16,), ch, jnp.int32) for ch in range(channels)]

            @plsc.parallel_loop(0, segs, unroll=2)
            def _seg(s):
                for ch in range(channels):
                    for jg in range(lanes // 16):
                        val = plsc.load_gather(
                            bi, [s + carries[jg], cvecs[ch], wmods[jg]]
                        )
                        bo[s, ch, pl.ds(jg * 16, 16)] = val

            out_handles[g] = out_copy(g)
        out_handles[nsteps - 2].wait()
        out_handles[nsteps - 1].wait()

    return shift_kernel


def kernel(wav):
    batch, sources, channels, time = wav.shape
    length = time - _SHIFT
    lanes = 128
    in_tb = time // lanes
    out_tb = length // lanes
    nblocks = batch * sources

    offs_key = jax.random.key(42)
    offsets = jax.random.randint(offs_key, (batch, sources, 1, 1), 0, _SHIFT)
    offs = offsets.reshape(nblocks).astype(jnp.int32)
    offs = jnp.pad(offs, (0, 16))

    blocks_per_worker = nblocks // _NUM_WORKERS
    x = wav.reshape(batch, sources, channels, in_tb, lanes)
    x = x.transpose(0, 1, 3, 2, 4).reshape(nblocks, in_tb, channels, lanes)
    out = _build_shift_kernel(
        nblocks, in_tb, out_tb, channels, lanes, blocks_per_worker
    )(x, offs)
    out = out.reshape(batch, sources, out_tb, channels, lanes)
    out = out.transpose(0, 1, 3, 2, 4).reshape(batch, sources, channels, length)
    return out


# speedup vs baseline: 17.8785x; 1.0506x over previous
import functools

import jax
import jax.numpy as jnp
from jax import lax
from jax.experimental import pallas as pl
from jax.experimental.pallas import tpu as pltpu
from jax.experimental.pallas import tpu_sc as plsc

_SHIFT = 8192
_NUM_WORKERS = 32
_SEGS = 120


def _build_shift_kernel(nblocks, in_tb, out_tb, channels, lanes, blocks_per_worker):
    mesh = plsc.VectorSubcoreMesh(core_axis_name="c", subcore_axis_name="s")
    segs = _SEGS
    assert out_tb % segs == 0
    nchunks = out_tb // segs
    nsteps = blocks_per_worker * nchunks

    @functools.partial(
        pl.kernel,
        mesh=mesh,
        out_type=jax.ShapeDtypeStruct((nblocks, out_tb, channels, lanes), jnp.float32),
        scratch_types=[
            pltpu.VMEM((nblocks + 16,), jnp.int32),
            pltpu.VMEM((segs + 1, channels, lanes), jnp.float32),
            pltpu.VMEM((segs + 1, channels, lanes), jnp.float32),
            pltpu.VMEM((segs, channels, lanes), jnp.float32),
            pltpu.VMEM((segs, channels, lanes), jnp.float32),
            pltpu.SemaphoreType.DMA,
            pltpu.SemaphoreType.DMA,
            pltpu.SemaphoreType.DMA,
            pltpu.SemaphoreType.DMA,
        ],
        compiler_params=pltpu.CompilerParams(
            use_tc_tiling_on_sc=False, needs_layout_passes=False
        ),
    )
    def shift_kernel(
        x_hbm, offs_hbm, out_hbm,
        offs_v, bin0, bin1, bout0, bout1, si0, si1, so0, so1,
    ):
        wid = lax.axis_index("s") * 2 + lax.axis_index("c")
        base_blk = wid * blocks_per_worker
        pltpu.sync_copy(offs_hbm, offs_v)
        offs_vec = offs_v[pl.ds(base_blk, 16)]
        lane = jax.lax.iota(jnp.int32, 16)
        qs, rems = [], []
        for b in range(blocks_per_worker):
            off = offs_vec[b]
            q = off // lanes
            qs.append(q)
            rems.append(off - q * lanes)
        bins = [bin0, bin1]
        bouts = [bout0, bout1]
        isems = [si0, si1]
        osems = [so0, so1]

        def in_copy(g):
            b, c = divmod(g, nchunks)
            return pltpu.async_copy(
                x_hbm.at[base_blk + b, pl.ds(qs[b] + c * segs, segs + 1), :, :],
                bins[g % 2],
                isems[g % 2],
            )

        def out_copy(g):
            b, c = divmod(g, nchunks)
            return pltpu.async_copy(
                bouts[g % 2],
                out_hbm.at[base_blk + b, pl.ds(c * segs, segs), :, :],
                osems[g % 2],
            )

        in_handles = {0: in_copy(0)}
        out_handles = {}
        for g in range(nsteps):
            if g + 1 < nsteps:
                in_handles[g + 1] = in_copy(g + 1)
            in_handles[g].wait()
            if g >= 2:
                out_handles[g - 2].wait()
            b, c = divmod(g, nchunks)
            rem = rems[b]
            bi = bins[g % 2]
            bo = bouts[g % 2]
            carries, wmods = [], []
            lane_bits = lanes.bit_length() - 1
            for jg in range(lanes // 16):
                w = rem + jg * 16 + lane
                carries.append(lax.shift_right_logical(w, lane_bits))
                wmods.append(jnp.bitwise_and(w, lanes - 1))
            cvecs = [jnp.full((16,), ch, jnp.int32) for ch in range(channels)]

            @plsc.parallel_loop(0, segs, unroll=4)
            def _seg(s):
                bis = bi.at[pl.ds(s, 2)]
                for ch in range(channels):
                    for jg in range(lanes // 16):
                        val = plsc.load_gather(
                            bis, [carries[jg], cvecs[ch], wmods[jg]]
                        )
                        bo[s, ch, pl.ds(jg * 16, 16)] = val

            out_handles[g] = out_copy(g)
        out_handles[nsteps - 2].wait()
        out_handles[nsteps - 1].wait()

    return shift_kernel


def kernel(wav):
    batch, sources, channels, time = wav.shape
    length = time - _SHIFT
    lanes = 128
    in_tb = time // lanes
    out_tb = length // lanes
    nblocks = batch * sources

    def _make_offs():
        offs_key = jax.random.key(42)
        offsets = jax.random.randint(offs_key, (batch, sources, 1, 1), 0, _SHIFT)
        o = offsets.reshape(nblocks).astype(jnp.int32)
        return jnp.pad(o, (0, 16))

    try:
        import numpy as np

        _cpu = jax.local_devices(backend="cpu")[0]
        with jax.ensure_compile_time_eval(), jax.default_device(_cpu):
            offs = jnp.asarray(np.asarray(_make_offs()))
    except Exception:
        offs = _make_offs()

    blocks_per_worker = nblocks // _NUM_WORKERS
    x = wav.reshape(batch, sources, channels, in_tb, lanes)
    x = x.transpose(0, 1, 3, 2, 4).reshape(nblocks, in_tb, channels, lanes)
    out = _build_shift_kernel(
        nblocks, in_tb, out_tb, channels, lanes, blocks_per_worker
    )(x, offs)
    out = out.reshape(batch, sources, out_tb, channels, lanes)
    out = out.transpose(0, 1, 3, 2, 4).reshape(batch, sources, channels, length)
    return out
